# baseline (speedup 1.0000x reference)
import functools

import jax
import jax.numpy as jnp
from jax import lax
from jax.experimental import pallas as pl
from jax.experimental.pallas import tpu as pltpu
from jax.experimental.pallas import tpu_sc as plsc

KCH = 64
PREC = 2 * KCH
LN_EPS = 1e-5


def _transform_body(x_ref, w_ref, y_ref):
    y_ref[...] = jnp.dot(x_ref[...], w_ref[0],
                         preferred_element_type=jnp.float32)


def _transform(x, w_ext, bn):
    n, c = x.shape
    k = w_ext.shape[0]
    nb = n // bn
    return pl.pallas_call(
        _transform_body,
        grid=(nb, k),
        in_specs=[
            pl.BlockSpec((bn, c), lambda i, r: (i, 0)),
            pl.BlockSpec((1, c, c), lambda i, r: (r, 0, 0)),
        ],
        out_specs=pl.BlockSpec((bn, c), lambda i, r, _nb=nb: (r * _nb + i, 0)),
        out_shape=jax.ShapeDtypeStruct((k * n, c), jnp.float32),
    )(x, w_ext)


def _mid_body(acc_ref, r1_ref, b_ref, g_ref, be_ref, w_ref, y_ref):
    s = acc_ref[0] + acc_ref[1] + r1_ref[...] + b_ref[...]
    mu = jnp.mean(s, axis=-1, keepdims=True)
    var = jnp.mean((s - mu) ** 2, axis=-1, keepdims=True)
    h = (s - mu) / jnp.sqrt(var + LN_EPS) * g_ref[...] + be_ref[...]
    h = jnp.maximum(h, 0.0)
    y_ref[...] = jnp.dot(h, w_ref[0], preferred_element_type=jnp.float32)


def _mid(acc, y1, b1, g1, be1, w2_ext, n, bn):
    c = y1.shape[1]
    k = w2_ext.shape[0]
    nb = n // bn
    return pl.pallas_call(
        _mid_body,
        grid=(nb, k),
        in_specs=[
            pl.BlockSpec((2, bn, c), lambda i, r: (0, i, 0)),
            pl.BlockSpec((bn, c), lambda i, r, _nb=nb, _k=k: ((_k - 1) * _nb + i, 0)),
            pl.BlockSpec((c,), lambda i, r: (0,)),
            pl.BlockSpec((c,), lambda i, r: (0,)),
            pl.BlockSpec((c,), lambda i, r: (0,)),
            pl.BlockSpec((1, c, c), lambda i, r: (r, 0, 0)),
        ],
        out_specs=pl.BlockSpec((bn, c), lambda i, r, _nb=nb: (r * _nb + i, 0)),
        out_shape=jax.ShapeDtypeStruct((k * n, c), jnp.float32),
    )(acc, y1, b1, g1, be1, w2_ext)


def _final_body(acc_ref, r2_ref, b_ref, out_ref):
    out_ref[...] = acc_ref[0] + acc_ref[1] + r2_ref[...] + b_ref[...]


def _final(acc, y2, b2, n, bn):
    c = y2.shape[1]
    k = y2.shape[0] // n
    nb = n // bn
    return pl.pallas_call(
        _final_body,
        grid=(nb,),
        in_specs=[
            pl.BlockSpec((2, bn, c), lambda i: (0, i, 0)),
            pl.BlockSpec((bn, c), lambda i, _nb=nb, _k=k: ((_k - 1) * _nb + i, 0)),
            pl.BlockSpec((c,), lambda i: (0,)),
        ],
        out_specs=pl.BlockSpec((bn, c), lambda i: (i, 0)),
        out_shape=jax.ShapeDtypeStruct((n, c), jnp.float32),
    )(acc, y2, b2)


def _zero_vec(ref, nwords):
    def st(j, _):
        ref[pl.ds(j * 16, 16)] = jnp.zeros((16,), jnp.float32)
        return 0
    lax.fori_loop(0, nwords // 16, st, 0)


def _fill_ones(ref, nwords):
    def st(j, _):
        ref[pl.ds(j * 16, 16)] = jnp.ones((16,), jnp.float32)
        return 0
    lax.fori_loop(0, nwords // 16, st, 0)


def _sc_prep_body(src_hbm, dst_hbm, et_hbm, packed_hbm, wts_hbm,
                  cnt_sh, sbuf, dbuf, tbuf, ibuf, pbuf, wfbuf, ones_v,
                  zbuf, sem, *, n, rnp, nchunks, nc, ns):
    core = lax.axis_index("c")
    sid = lax.axis_index("s")
    wid = core * ns + sid
    nw = nc * ns
    csl = rnp // ns

    _zero_vec(zbuf, csl)
    _fill_ones(ones_v, KCH)
    tsl = pl.ds(sid * csl, csl)
    pltpu.sync_copy(zbuf, cnt_sh.at[tsl])
    plsc.subcore_barrier()

    nk_c = nchunks // ns

    def count_chunk(k_, _):
        base = (sid + k_ * ns) * KCH
        pltpu.sync_copy(dst_hbm.at[pl.ds(base, KCH)], dbuf)
        pltpu.sync_copy(et_hbm.at[pl.ds(base, KCH)], tbuf)

        def cidx(j, _):
            sl = pl.ds(j * 16, 16)
            ibuf[sl] = tbuf[sl] * n + dbuf[sl]
            return 0
        lax.fori_loop(0, KCH // 16, cidx, 0)
        pltpu.sync_copy(ones_v, cnt_sh.at[ibuf], add=True)
        return 0

    lax.fori_loop(0, nk_c, count_chunk, 0)
    plsc.subcore_barrier()

    pltpu.sync_copy(cnt_sh.at[tsl], zbuf)

    def invb(j, _):
        sl = pl.ds(j * 16, 16)
        zbuf[sl] = 1.0 / jnp.maximum(zbuf[sl], 1.0)
        return 0
    lax.fori_loop(0, csl // 16, invb, 0)
    pltpu.sync_copy(zbuf, cnt_sh.at[tsl])
    plsc.subcore_barrier()

    nk = nchunks // nw

    def pack_chunk(j, _):
        k = wid + j * nw
        base = k * KCH
        pltpu.sync_copy(src_hbm.at[pl.ds(base, KCH)], sbuf)
        pltpu.sync_copy(dst_hbm.at[pl.ds(base, KCH)], dbuf)
        pltpu.sync_copy(et_hbm.at[pl.ds(base, KCH)], tbuf)

        def gw(g, _):
            sl = pl.ds(g * 16, 16)
            t = tbuf[sl]
            pbuf[sl] = t * n + sbuf[sl]
            pbuf[pl.ds(KCH + g * 16, 16)] = dbuf[sl]
            ibuf[sl] = t * n + dbuf[sl]
            return 0
        lax.fori_loop(0, KCH // 16, gw, 0)

        pltpu.async_copy(cnt_sh.at[ibuf], wfbuf, sem).wait()
        pltpu.sync_copy(pbuf, packed_hbm.at[pl.ds(k * PREC, PREC)])
        pltpu.sync_copy(wfbuf, wts_hbm.at[pl.ds(k * KCH, KCH)])
        return 0

    lax.fori_loop(0, nk, pack_chunk, 0)


def _scale_rows(rv, wf, c):
    cl = c // 16

    def scale(g, _):
        w16 = wf[pl.ds(g * 16, 16)]
        for e_ in range(16):
            i = g * 16 + e_
            w = w16[e_]
            for j in range(cl):
                sl = pl.ds(j * 16, 16)
                rv[i, sl] = rv[i, sl] * w
        return 0
    lax.fori_loop(0, KCH // 16, scale, 0)


def _sc_value_body(packed_hbm, wts_hbm, y_hbm, acc_hbm, acc_sh,
                   p0, p1, p2, p3, w0, w1, w2, w3, r0, r1, r2, r3,
                   g0, g1, g2, g3, s0, s1, s2, s3,
                   *, np_, nchunks, c, nc, ns):
    core = lax.axis_index("c")
    sid = lax.axis_index("s")
    wid = core * ns + sid
    nw = nc * ns
    nk = nchunks // nw
    asl = np_ // ns

    P = (p0, p1, p2, p3)
    WF = (w0, w1, w2, w3)
    R = (r0, r1, r2, r3)
    G = (g0, g1, g2, g3)
    S = (s0, s1, s2, s3)

    def zr(i, _):
        for j in range(c // 16):
            r0[i, pl.ds(j * 16, 16)] = jnp.zeros((16,), jnp.float32)
        return 0
    lax.fori_loop(0, KCH, zr, 0)
    for z in range(asl // KCH):
        pltpu.sync_copy(r0, acc_sh.at[pl.ds(sid * asl + z * KCH, KCH)])
    plsc.subcore_barrier()

    def load_gather(m, s):
        k = wid + m * nw
        pltpu.sync_copy(packed_hbm.at[pl.ds(k * PREC, PREC)], P[s])
        pltpu.sync_copy(wts_hbm.at[pl.ds(k * KCH, KCH)], WF[s])
        pltpu.async_copy(y_hbm.at[P[s].at[pl.ds(0, KCH)]], R[s], G[s])

    def wait_gather(s):
        pltpu.make_async_copy(y_hbm.at[P[s].at[pl.ds(0, KCH)]],
                              R[s], G[s]).wait()

    def start_scatter(s):
        pltpu.async_copy(R[s], acc_sh.at[P[s].at[pl.ds(KCH, KCH)]], S[s],
                         add=True)

    def wait_scatter(s):
        pltpu.make_async_copy(R[s], acc_sh.at[P[s].at[pl.ds(KCH, KCH)]],
                              S[s]).wait()

    def process(s):
        wait_gather(s)
        _scale_rows(R[s], WF[s], c)
        start_scatter(s)

    load_gather(0, 0)
    load_gather(1, 1)
    process(0)
    load_gather(2, 2)
    process(1)
    load_gather(3, 3)
    process(2)
    wait_scatter(0)
    load_gather(4, 0)
    process(3)
    wait_scatter(1)
    load_gather(5, 1)

    nr = nk // 4

    def round_body(r_, _):
        for s in range(4):
            k = r_ * 4 + s
            process(s)
            m = k + 2
            t = (s + 2) % 4

            @pl.when(m < nk)
            def _():
                wait_scatter(t)
                load_gather(m, t)
        return 0

    lax.fori_loop(1, nr, round_body, 0)

    for s in range(4):
        wait_scatter(s)
    plsc.subcore_barrier()

    for z in range(asl // KCH):
        row = sid * asl + z * KCH
        pltpu.sync_copy(acc_sh.at[pl.ds(row, KCH)], r0)
        pltpu.sync_copy(r0, acc_hbm.at[core, pl.ds(row, KCH)])


def kernel(x, edge_index, edge_type, W1, root1, b1, gamma1, beta1,
           W2, root2, b2):
    n, c = x.shape
    e = edge_type.shape[0]
    r = W1.shape[0]

    info = plsc.get_sparse_core_info()
    nc, ns = info.num_cores, info.num_subcores
    nw = nc * ns

    asl = ((-(-n // ns)) + KCH - 1) // KCH * KCH
    np_ = asl * ns
    csl = (-(-(r * n) // ns) + 15) // 16 * 16
    rnp = csl * ns
    nk = max(8, -(-(-(-e // (nw * KCH))) // 4) * 4)
    nchunks = nk * nw
    e_pad = nchunks * KCH

    src = edge_index[0].astype(jnp.int32)
    dst = edge_index[1].astype(jnp.int32)
    et = edge_type.astype(jnp.int32)
    if e_pad > e:
        pad = e_pad - e
        src = jnp.concatenate([src, jnp.zeros((pad,), jnp.int32)])
        dst = jnp.concatenate([dst, jnp.full((pad,), n, jnp.int32)])
        et = jnp.concatenate([et, jnp.full((pad,), r - 1, jnp.int32)])

    w1_ext = jnp.concatenate([W1, root1[None]], axis=0)
    w2_ext = jnp.concatenate([W2, root2[None]], axis=0)

    mesh = plsc.VectorSubcoreMesh(core_axis_name="c", subcore_axis_name="s")
    f32 = jnp.float32
    i32 = jnp.int32

    prep = pl.kernel(
        functools.partial(_sc_prep_body, n=n, rnp=rnp, nchunks=nchunks,
                          nc=nc, ns=ns),
        out_type=[jax.ShapeDtypeStruct((nchunks * PREC,), i32),
                  jax.ShapeDtypeStruct((nchunks * KCH,), f32)],
        mesh=mesh,
        scratch_types=[
            pltpu.VMEM_SHARED((rnp,), f32),
            pltpu.VMEM((KCH,), i32),
            pltpu.VMEM((KCH,), i32),
            pltpu.VMEM((KCH,), i32),
            pltpu.VMEM((KCH,), i32),
            pltpu.VMEM((PREC,), i32),
            pltpu.VMEM((KCH,), f32),
            pltpu.VMEM((KCH,), f32),
            pltpu.VMEM((rnp // ns,), f32),
            pltpu.SemaphoreType.DMA,
        ],
    )
    packed, wts = prep(src, dst, et)

    bn = 400 if n % 400 == 0 else 100
    y1 = _transform(x, w1_ext, bn)

    value_scratch = [
        pltpu.VMEM_SHARED((np_, c), f32),
        pltpu.VMEM((PREC,), i32),
        pltpu.VMEM((PREC,), i32),
        pltpu.VMEM((PREC,), i32),
        pltpu.VMEM((PREC,), i32),
        pltpu.VMEM((KCH,), f32),
        pltpu.VMEM((KCH,), f32),
        pltpu.VMEM((KCH,), f32),
        pltpu.VMEM((KCH,), f32),
        pltpu.VMEM((KCH, c), f32),
        pltpu.VMEM((KCH, c), f32),
        pltpu.VMEM((KCH, c), f32),
        pltpu.VMEM((KCH, c), f32),
        pltpu.SemaphoreType.DMA,
        pltpu.SemaphoreType.DMA,
        pltpu.SemaphoreType.DMA,
        pltpu.SemaphoreType.DMA,
        pltpu.SemaphoreType.DMA,
        pltpu.SemaphoreType.DMA,
        pltpu.SemaphoreType.DMA,
        pltpu.SemaphoreType.DMA,
    ]

    value = pl.kernel(
        functools.partial(_sc_value_body, np_=np_, nchunks=nchunks,
                          c=c, nc=nc, ns=ns),
        out_type=jax.ShapeDtypeStruct((2, np_, c), f32),
        mesh=mesh,
        scratch_types=value_scratch,
    )

    acc1 = value(packed, wts, y1)
    y2 = _mid(acc1, y1, b1, gamma1, beta1, w2_ext, n, bn)
    acc2 = value(packed, wts, y2)
    return _final(acc2, y2, b2, n, bn)


# speedup vs baseline: 1.1124x; 1.1124x over previous
import functools

import jax
import jax.numpy as jnp
from jax import lax
from jax.experimental import pallas as pl
from jax.experimental.pallas import tpu as pltpu
from jax.experimental.pallas import tpu_sc as plsc

KCH = 128
LN_EPS = 1e-5


def _transform_body(x_ref, w_ref, y_ref):
    y_ref[...] = jnp.dot(x_ref[...], w_ref[0],
                         preferred_element_type=jnp.float32)


def _transform(x, w_ext, bn):
    n, c = x.shape
    k = w_ext.shape[0]
    nb = n // bn
    return pl.pallas_call(
        _transform_body,
        grid=(nb, k),
        in_specs=[
            pl.BlockSpec((bn, c), lambda i, r: (i, 0)),
            pl.BlockSpec((1, c, c), lambda i, r: (r, 0, 0)),
        ],
        out_specs=pl.BlockSpec((bn, c), lambda i, r, _nb=nb: (r * _nb + i, 0)),
        out_shape=jax.ShapeDtypeStruct((k * n, c), jnp.float32),
    )(x, w_ext)


def _mid_body(acc_ref, r1_ref, b_ref, g_ref, be_ref, w_ref, y_ref):
    s = acc_ref[0] + acc_ref[1] + r1_ref[...] + b_ref[...]
    mu = jnp.mean(s, axis=-1, keepdims=True)
    var = jnp.mean((s - mu) ** 2, axis=-1, keepdims=True)
    h = (s - mu) / jnp.sqrt(var + LN_EPS) * g_ref[...] + be_ref[...]
    h = jnp.maximum(h, 0.0)
    y_ref[...] = jnp.dot(h, w_ref[0], preferred_element_type=jnp.float32)


def _mid(acc, y1, b1, g1, be1, w2_ext, n, bn):
    c = y1.shape[1]
    k = w2_ext.shape[0]
    nb = n // bn
    return pl.pallas_call(
        _mid_body,
        grid=(nb, k),
        in_specs=[
            pl.BlockSpec((2, bn, c), lambda i, r: (0, i, 0)),
            pl.BlockSpec((bn, c), lambda i, r, _nb=nb, _k=k: ((_k - 1) * _nb + i, 0)),
            pl.BlockSpec((c,), lambda i, r: (0,)),
            pl.BlockSpec((c,), lambda i, r: (0,)),
            pl.BlockSpec((c,), lambda i, r: (0,)),
            pl.BlockSpec((1, c, c), lambda i, r: (r, 0, 0)),
        ],
        out_specs=pl.BlockSpec((bn, c), lambda i, r, _nb=nb: (r * _nb + i, 0)),
        out_shape=jax.ShapeDtypeStruct((k * n, c), jnp.float32),
    )(acc, y1, b1, g1, be1, w2_ext)


def _final_body(acc_ref, r2_ref, b_ref, out_ref):
    out_ref[...] = acc_ref[0] + acc_ref[1] + r2_ref[...] + b_ref[...]


def _final(acc, y2, b2, n, bn):
    c = y2.shape[1]
    k = y2.shape[0] // n
    nb = n // bn
    return pl.pallas_call(
        _final_body,
        grid=(nb,),
        in_specs=[
            pl.BlockSpec((2, bn, c), lambda i: (0, i, 0)),
            pl.BlockSpec((bn, c), lambda i, _nb=nb, _k=k: ((_k - 1) * _nb + i, 0)),
            pl.BlockSpec((c,), lambda i: (0,)),
        ],
        out_specs=pl.BlockSpec((bn, c), lambda i: (i, 0)),
        out_shape=jax.ShapeDtypeStruct((n, c), jnp.float32),
    )(acc, y2, b2)


def _zero_vec(ref, nwords):
    def st(j, _):
        ref[pl.ds(j * 16, 16)] = jnp.zeros((16,), jnp.float32)
        return 0
    lax.fori_loop(0, nwords // 16, st, 0)


def _fill_ones(ref, nwords):
    def st(j, _):
        ref[pl.ds(j * 16, 16)] = jnp.ones((16,), jnp.float32)
        return 0
    lax.fori_loop(0, nwords // 16, st, 0)


def _scale_rows(rv, wf, c):
    cl = c // 16

    def scale(g, _):
        w16 = wf[pl.ds(g * 16, 16)]
        for e_ in range(16):
            i = g * 16 + e_
            w = w16[e_]
            for j in range(cl):
                sl = pl.ds(j * 16, 16)
                rv[i, sl] = rv[i, sl] * w
        return 0
    lax.fori_loop(0, KCH // 16, scale, 0)


def _agg_value_pass(src_hbm, dst_hbm, et_hbm, y_hbm, acc_sh, inv_sh,
                    r0, r1, i0, i1, d0, d1, w0, w1, sbuf, tbuf,
                    g0, g1, s0, s1, wsem, wid, nw, n, nchunks, c):
    nk = nchunks // nw
    RR = (r0, r1)
    II = (i0, i1)
    DD = (d0, d1)
    WW = (w0, w1)
    GG = (g0, g1)
    SS = (s0, s1)

    def prep_chunk(m, s):
        base = (wid + m * nw) * KCH
        pltpu.sync_copy(src_hbm.at[pl.ds(base, KCH)], sbuf)
        pltpu.sync_copy(dst_hbm.at[pl.ds(base, KCH)], DD[s])
        pltpu.sync_copy(et_hbm.at[pl.ds(base, KCH)], tbuf)

        def gw(j, _):
            sl = pl.ds(j * 16, 16)
            t = tbuf[sl]
            II[s][sl] = t * n + sbuf[sl]
            sbuf[sl] = t * n + DD[s][sl]
            return 0
        lax.fori_loop(0, KCH // 16, gw, 0)

        pltpu.async_copy(inv_sh.at[sbuf], WW[s], wsem).wait()
        pltpu.async_copy(y_hbm.at[II[s]], RR[s], GG[s])

    def wait_g(s):
        pltpu.make_async_copy(y_hbm.at[II[s]], RR[s], GG[s]).wait()

    def start_sc(s):
        pltpu.async_copy(RR[s], acc_sh.at[DD[s]], SS[s], add=True)

    def wait_sc(s):
        pltpu.make_async_copy(RR[s], acc_sh.at[DD[s]], SS[s]).wait()

    prep_chunk(0, 0)
    wait_g(0)
    prep_chunk(1, 1)
    _scale_rows(r0, w0, c)
    start_sc(0)

    def pair(r_, _):
        k1 = 2 * r_ + 1

        wait_g(1)

        @pl.when(k1 + 1 < nk)
        def _():
            wait_sc(0)
            prep_chunk(k1 + 1, 0)

        _scale_rows(r1, w1, c)
        start_sc(1)

        k2 = k1 + 1

        @pl.when(k2 < nk)
        def _():
            wait_g(0)

            @pl.when(k2 + 1 < nk)
            def _():
                wait_sc(1)
                prep_chunk(k2 + 1, 1)

            _scale_rows(r0, w0, c)
            start_sc(0)
        return 0

    lax.fori_loop(0, nk // 2, pair, 0)
    wait_sc(0)
    wait_sc(1)


def _acc_writeout(acc_sh, rows_v, acc_hbm, core, sid, asl, c):
    for z in range(asl // KCH):
        row = sid * asl + z * KCH
        pltpu.sync_copy(acc_sh.at[pl.ds(row, KCH)], rows_v)
        pltpu.sync_copy(rows_v, acc_hbm.at[core, pl.ds(row, KCH)])


def _zero_acc(acc_sh, rows_v, sid, asl, c):
    def zr(i, _):
        for j in range(c // 16):
            rows_v[i, pl.ds(j * 16, 16)] = jnp.zeros((16,), jnp.float32)
        return 0
    lax.fori_loop(0, KCH, zr, 0)
    for z in range(asl // KCH):
        pltpu.sync_copy(rows_v, acc_sh.at[pl.ds(sid * asl + z * KCH, KCH)])


def _sc_first_body(src_hbm, dst_hbm, et_hbm, y_hbm, acc_hbm, inv_hbm,
                   cnt_sh, acc_sh, r0, r1, i0, i1, d0, d1, w0, w1,
                   sbuf, tbuf, ones_v, zbuf,
                   g0, g1, s0, s1, wsem, *, n, np_, rnp, nchunks, c,
                   nc, ns):
    core = lax.axis_index("c")
    sid = lax.axis_index("s")
    wid = core * ns + sid
    nw = nc * ns
    csl = rnp // ns
    asl = np_ // ns

    _zero_vec(zbuf, csl)
    _fill_ones(ones_v, KCH)
    pltpu.sync_copy(zbuf, cnt_sh.at[pl.ds(sid * csl, csl)])
    _zero_acc(acc_sh, r0, sid, asl, c)
    plsc.subcore_barrier()

    nk_c = nchunks // ns

    def count_chunk(k_, _):
        base = (sid + k_ * ns) * KCH
        pltpu.sync_copy(dst_hbm.at[pl.ds(base, KCH)], d0)
        pltpu.sync_copy(et_hbm.at[pl.ds(base, KCH)], tbuf)

        def cidx(j, _):
            sl = pl.ds(j * 16, 16)
            i0[sl] = tbuf[sl] * n + d0[sl]
            return 0
        lax.fori_loop(0, KCH // 16, cidx, 0)
        pltpu.sync_copy(ones_v, cnt_sh.at[i0], add=True)
        return 0

    lax.fori_loop(0, nk_c, count_chunk, 0)
    plsc.subcore_barrier()

    tsl = pl.ds(sid * csl, csl)
    pltpu.sync_copy(cnt_sh.at[tsl], zbuf)

    def invb(j, _):
        sl = pl.ds(j * 16, 16)
        zbuf[sl] = 1.0 / jnp.maximum(zbuf[sl], 1.0)
        return 0
    lax.fori_loop(0, csl // 16, invb, 0)
    pltpu.sync_copy(zbuf, cnt_sh.at[tsl])

    @pl.when(core == 0)
    def _():
        pltpu.sync_copy(zbuf, inv_hbm.at[tsl])

    plsc.subcore_barrier()

    _agg_value_pass(src_hbm, dst_hbm, et_hbm, y_hbm, acc_sh, cnt_sh,
                    r0, r1, i0, i1, d0, d1, w0, w1, sbuf, tbuf,
                    g0, g1, s0, s1, wsem, wid, nw, n, nchunks, c)
    plsc.subcore_barrier()

    _acc_writeout(acc_sh, r0, acc_hbm, core, sid, asl, c)


def _sc_second_body(src_hbm, dst_hbm, et_hbm, y_hbm, inv_hbm, acc_hbm,
                    inv_sh, acc_sh, r0, r1, i0, i1, d0, d1, w0, w1,
                    sbuf, tbuf, cbuf,
                    g0, g1, s0, s1, wsem, *, n, np_, rnp, nchunks, c,
                    nc, ns):
    core = lax.axis_index("c")
    sid = lax.axis_index("s")
    wid = core * ns + sid
    nw = nc * ns
    asl = np_ // ns
    csl = rnp // ns

    _zero_acc(acc_sh, r0, sid, asl, c)
    tsl = pl.ds(sid * csl, csl)
    pltpu.sync_copy(inv_hbm.at[tsl], cbuf)
    pltpu.sync_copy(cbuf, inv_sh.at[tsl])
    plsc.subcore_barrier()

    _agg_value_pass(src_hbm, dst_hbm, et_hbm, y_hbm, acc_sh, inv_sh,
                    r0, r1, i0, i1, d0, d1, w0, w1, sbuf, tbuf,
                    g0, g1, s0, s1, wsem, wid, nw, n, nchunks, c)
    plsc.subcore_barrier()

    _acc_writeout(acc_sh, r0, acc_hbm, core, sid, asl, c)


def kernel(x, edge_index, edge_type, W1, root1, b1, gamma1, beta1,
           W2, root2, b2):
    n, c = x.shape
    e = edge_type.shape[0]
    r = W1.shape[0]

    info = plsc.get_sparse_core_info()
    nc, ns = info.num_cores, info.num_subcores
    nw = nc * ns

    asl = ((-(-n // ns)) + KCH - 1) // KCH * KCH
    np_ = asl * ns
    csl = (-(-(r * n) // ns) + 15) // 16 * 16
    rnp = csl * ns
    nk = max(2, -(-(-(-e // (nw * KCH))) // 2) * 2)
    nchunks = nk * nw
    e_pad = nchunks * KCH

    src = edge_index[0].astype(jnp.int32)
    dst = edge_index[1].astype(jnp.int32)
    et = edge_type.astype(jnp.int32)
    if e_pad > e:
        pad = e_pad - e
        src = jnp.concatenate([src, jnp.zeros((pad,), jnp.int32)])
        dst = jnp.concatenate([dst, jnp.full((pad,), n, jnp.int32)])
        et = jnp.concatenate([et, jnp.full((pad,), r - 1, jnp.int32)])

    w1_ext = jnp.concatenate([W1, root1[None]], axis=0)
    w2_ext = jnp.concatenate([W2, root2[None]], axis=0)

    bn = 400 if n % 400 == 0 else 100
    y1 = _transform(x, w1_ext, bn)

    mesh = plsc.VectorSubcoreMesh(core_axis_name="c", subcore_axis_name="s")
    f32 = jnp.float32
    i32 = jnp.int32

    common_scratch = [
        pltpu.VMEM((KCH, c), f32),
        pltpu.VMEM((KCH, c), f32),
        pltpu.VMEM((KCH,), i32),
        pltpu.VMEM((KCH,), i32),
        pltpu.VMEM((KCH,), i32),
        pltpu.VMEM((KCH,), i32),
        pltpu.VMEM((KCH,), f32),
        pltpu.VMEM((KCH,), f32),
        pltpu.VMEM((KCH,), i32),
        pltpu.VMEM((KCH,), i32),
    ]
    sems = [
        pltpu.SemaphoreType.DMA,
        pltpu.SemaphoreType.DMA,
        pltpu.SemaphoreType.DMA,
        pltpu.SemaphoreType.DMA,
        pltpu.SemaphoreType.DMA,
    ]

    first = pl.kernel(
        functools.partial(_sc_first_body, n=n, np_=np_, rnp=rnp,
                          nchunks=nchunks, c=c, nc=nc, ns=ns),
        out_type=[jax.ShapeDtypeStruct((2, np_, c), f32),
                  jax.ShapeDtypeStruct((rnp,), f32)],
        mesh=mesh,
        scratch_types=[
            pltpu.VMEM_SHARED((rnp,), f32),
            pltpu.VMEM_SHARED((np_, c), f32),
            *common_scratch,
            pltpu.VMEM((KCH,), f32),
            pltpu.VMEM((rnp // ns,), f32),
            *sems,
        ],
    )
    acc1, inv = first(src, dst, et, y1)

    y2 = _mid(acc1, y1, b1, gamma1, beta1, w2_ext, n, bn)

    second = pl.kernel(
        functools.partial(_sc_second_body, n=n, np_=np_, rnp=rnp,
                          nchunks=nchunks, c=c, nc=nc, ns=ns),
        out_type=jax.ShapeDtypeStruct((2, np_, c), f32),
        mesh=mesh,
        scratch_types=[
            pltpu.VMEM_SHARED((rnp,), f32),
            pltpu.VMEM_SHARED((np_, c), f32),
            *common_scratch,
            pltpu.VMEM((rnp // ns,), f32),
            *sems,
        ],
    )
    acc2 = second(src, dst, et, y2, inv)

    return _final(acc2, y2, b2, n, bn)


# speedup vs baseline: 1.3341x; 1.1994x over previous
import functools

import jax
import jax.numpy as jnp
from jax import lax
from jax.experimental import pallas as pl
from jax.experimental.pallas import tpu as pltpu
from jax.experimental.pallas import tpu_sc as plsc

KCH = 128
LN_EPS = 1e-5


def _transform_body(x_ref, w_ref, y_ref):
    y_ref[...] = jnp.dot(x_ref[...], w_ref[0],
                         preferred_element_type=jnp.float32)


def _transform(x, w_ext, bn):
    n, c = x.shape
    k = w_ext.shape[0]
    nb = n // bn
    return pl.pallas_call(
        _transform_body,
        grid=(nb, k),
        in_specs=[
            pl.BlockSpec((bn, c), lambda i, r: (i, 0)),
            pl.BlockSpec((1, c, c), lambda i, r: (r, 0, 0)),
        ],
        out_specs=pl.BlockSpec((bn, c), lambda i, r, _nb=nb: (r * _nb + i, 0)),
        out_shape=jax.ShapeDtypeStruct((k * n, c), jnp.float32),
    )(x, w_ext)


def _mid_body(acc_ref, r1_ref, b_ref, g_ref, be_ref, w_ref, y_ref):
    s = acc_ref[0] + acc_ref[1] + r1_ref[...] + b_ref[...]
    mu = jnp.mean(s, axis=-1, keepdims=True)
    var = jnp.mean((s - mu) ** 2, axis=-1, keepdims=True)
    h = (s - mu) / jnp.sqrt(var + LN_EPS) * g_ref[...] + be_ref[...]
    h = jnp.maximum(h, 0.0)
    y_ref[...] = jnp.dot(h, w_ref[0], preferred_element_type=jnp.float32)


def _mid(acc, y1, b1, g1, be1, w2_ext, n, bn):
    c = y1.shape[1]
    k = w2_ext.shape[0]
    nb = n // bn
    return pl.pallas_call(
        _mid_body,
        grid=(nb, k),
        in_specs=[
            pl.BlockSpec((2, bn, c), lambda i, r: (0, i, 0)),
            pl.BlockSpec((bn, c), lambda i, r, _nb=nb, _k=k: ((_k - 1) * _nb + i, 0)),
            pl.BlockSpec((c,), lambda i, r: (0,)),
            pl.BlockSpec((c,), lambda i, r: (0,)),
            pl.BlockSpec((c,), lambda i, r: (0,)),
            pl.BlockSpec((1, c, c), lambda i, r: (r, 0, 0)),
        ],
        out_specs=pl.BlockSpec((bn, c), lambda i, r, _nb=nb: (r * _nb + i, 0)),
        out_shape=jax.ShapeDtypeStruct((k * n, c), jnp.float32),
    )(acc, y1, b1, g1, be1, w2_ext)


def _final_body(acc_ref, r2_ref, b_ref, out_ref):
    out_ref[...] = acc_ref[0] + acc_ref[1] + r2_ref[...] + b_ref[...]


def _final(acc, y2, b2, n, bn):
    c = y2.shape[1]
    k = y2.shape[0] // n
    nb = n // bn
    return pl.pallas_call(
        _final_body,
        grid=(nb,),
        in_specs=[
            pl.BlockSpec((2, bn, c), lambda i: (0, i, 0)),
            pl.BlockSpec((bn, c), lambda i, _nb=nb, _k=k: ((_k - 1) * _nb + i, 0)),
            pl.BlockSpec((c,), lambda i: (0,)),
        ],
        out_specs=pl.BlockSpec((bn, c), lambda i: (i, 0)),
        out_shape=jax.ShapeDtypeStruct((n, c), jnp.float32),
    )(acc, y2, b2)


def _zero_vec(ref, nwords):
    def st(j, _):
        ref[pl.ds(j * 16, 16)] = jnp.zeros((16,), jnp.float32)
        return 0
    lax.fori_loop(0, nwords // 16, st, 0)


def _fill_ones(ref, nwords):
    def st(j, _):
        ref[pl.ds(j * 16, 16)] = jnp.ones((16,), jnp.float32)
        return 0
    lax.fori_loop(0, nwords // 16, st, 0)


def _agg_value_pass(src_hbm, dst_hbm, et_hbm, y_hbm, acc_sh, inv_sh,
                    rows_v, sbuf, dbuf, tbuf, ibuf, widx, wbuf, sem,
                    wid, nw, n, nchunks, c):
    nk = nchunks // nw
    cl = c // 16

    def chunk(k_, _):
        base = (wid + k_ * nw) * KCH
        pltpu.sync_copy(src_hbm.at[pl.ds(base, KCH)], sbuf)
        pltpu.sync_copy(dst_hbm.at[pl.ds(base, KCH)], dbuf)
        pltpu.sync_copy(et_hbm.at[pl.ds(base, KCH)], tbuf)

        def gw(j, _):
            sl = pl.ds(j * 16, 16)
            t = tbuf[sl]
            ibuf[sl] = t * n + sbuf[sl]
            widx[sl] = t * n + dbuf[sl]
            return 0
        lax.fori_loop(0, KCH // 16, gw, 0)

        pltpu.async_copy(y_hbm.at[ibuf], rows_v, sem).wait()
        pltpu.async_copy(inv_sh.at[widx], wbuf, sem).wait()

        def scale(g, _):
            w16 = wbuf[pl.ds(g * 16, 16)]
            for e_ in range(16):
                i = g * 16 + e_
                w = w16[e_]
                for j in range(cl):
                    sl = pl.ds(j * 16, 16)
                    rows_v[i, sl] = rows_v[i, sl] * w
            return 0
        lax.fori_loop(0, KCH // 16, scale, 0)

        pltpu.sync_copy(rows_v, acc_sh.at[dbuf], add=True)
        return 0

    lax.fori_loop(0, nk, chunk, 0)


def _acc_writeout(acc_sh, rows_v, acc_hbm, core, sid, asl, c):
    for z in range(asl // KCH):
        row = sid * asl + z * KCH
        pltpu.sync_copy(acc_sh.at[pl.ds(row, KCH)], rows_v)
        pltpu.sync_copy(rows_v, acc_hbm.at[core, pl.ds(row, KCH)])


def _sc_hist_body(dst_hbm, et_hbm, inv_hbm, cnt_sh, dbuf, tbuf, ibuf,
                  ones_v, zbuf, *, n, rnp, nchunks, nc, ns):
    core = lax.axis_index("c")
    sid = lax.axis_index("s")
    csl = rnp // ns

    _zero_vec(zbuf, csl)
    _fill_ones(ones_v, KCH)
    tsl = pl.ds(sid * csl, csl)
    pltpu.sync_copy(zbuf, cnt_sh.at[tsl])
    plsc.subcore_barrier()

    nk_c = nchunks // ns

    def count_chunk(k_, _):
        base = (sid + k_ * ns) * KCH
        pltpu.sync_copy(dst_hbm.at[pl.ds(base, KCH)], dbuf)
        pltpu.sync_copy(et_hbm.at[pl.ds(base, KCH)], tbuf)

        def cidx(j, _):
            sl = pl.ds(j * 16, 16)
            ibuf[sl] = tbuf[sl] * n + dbuf[sl]
            return 0
        lax.fori_loop(0, KCH // 16, cidx, 0)
        pltpu.sync_copy(ones_v, cnt_sh.at[ibuf], add=True)
        return 0

    @pl.when(core == 0)
    def _():
        lax.fori_loop(0, nk_c, count_chunk, 0)

    plsc.subcore_barrier()

    @pl.when(core == 0)
    def _():
        pltpu.sync_copy(cnt_sh.at[tsl], zbuf)

        def invb(j, _):
            sl = pl.ds(j * 16, 16)
            zbuf[sl] = 1.0 / jnp.maximum(zbuf[sl], 1.0)
            return 0
        lax.fori_loop(0, csl // 16, invb, 0)
        pltpu.sync_copy(zbuf, inv_hbm.at[tsl])


def _sc_value_body(src_hbm, dst_hbm, et_hbm, y_hbm, inv_hbm, acc_hbm,
                   inv_sh, acc_sh, rows_v, sbuf, dbuf, tbuf, ibuf, widx,
                   wbuf, cbuf, sem, *, n, np_, rnp, nchunks, c, nc, ns):
    core = lax.axis_index("c")
    sid = lax.axis_index("s")
    wid = core * ns + sid
    nw = nc * ns
    asl = np_ // ns
    csl = rnp // ns

    def zr(i, _):
        for j in range(c // 16):
            rows_v[i, pl.ds(j * 16, 16)] = jnp.zeros((16,), jnp.float32)
        return 0
    lax.fori_loop(0, KCH, zr, 0)
    for z in range(asl // KCH):
        pltpu.sync_copy(rows_v, acc_sh.at[pl.ds(sid * asl + z * KCH, KCH)])
    tsl = pl.ds(sid * csl, csl)
    pltpu.sync_copy(inv_hbm.at[tsl], cbuf)
    pltpu.sync_copy(cbuf, inv_sh.at[tsl])
    plsc.subcore_barrier()

    _agg_value_pass(src_hbm, dst_hbm, et_hbm, y_hbm, acc_sh, inv_sh,
                    rows_v, sbuf, dbuf, tbuf, ibuf, widx, wbuf, sem,
                    wid, nw, n, nchunks, c)
    plsc.subcore_barrier()

    _acc_writeout(acc_sh, rows_v, acc_hbm, core, sid, asl, c)


def kernel(x, edge_index, edge_type, W1, root1, b1, gamma1, beta1,
           W2, root2, b2):
    n, c = x.shape
    e = edge_type.shape[0]
    r = W1.shape[0]

    info = plsc.get_sparse_core_info()
    nc, ns = info.num_cores, info.num_subcores
    nw = nc * ns

    asl = ((-(-n // ns)) + KCH - 1) // KCH * KCH
    np_ = asl * ns
    csl = (-(-(r * n) // ns) + 15) // 16 * 16
    rnp = csl * ns
    nchunks = -(-e // (nw * KCH)) * nw
    e_pad = nchunks * KCH

    src = edge_index[0].astype(jnp.int32)
    dst = edge_index[1].astype(jnp.int32)
    et = edge_type.astype(jnp.int32)
    if e_pad > e:
        pad = e_pad - e
        src = jnp.concatenate([src, jnp.zeros((pad,), jnp.int32)])
        dst = jnp.concatenate([dst, jnp.full((pad,), n, jnp.int32)])
        et = jnp.concatenate([et, jnp.full((pad,), r - 1, jnp.int32)])

    w1_ext = jnp.concatenate([W1, root1[None]], axis=0)
    w2_ext = jnp.concatenate([W2, root2[None]], axis=0)

    mesh = plsc.VectorSubcoreMesh(core_axis_name="c", subcore_axis_name="s")
    f32 = jnp.float32
    i32 = jnp.int32

    hist = pl.kernel(
        functools.partial(_sc_hist_body, n=n, rnp=rnp, nchunks=nchunks,
                          nc=nc, ns=ns),
        out_type=jax.ShapeDtypeStruct((rnp,), f32),
        mesh=mesh,
        scratch_types=[
            pltpu.VMEM_SHARED((rnp,), f32),
            pltpu.VMEM((KCH,), i32),
            pltpu.VMEM((KCH,), i32),
            pltpu.VMEM((KCH,), i32),
            pltpu.VMEM((KCH,), f32),
            pltpu.VMEM((rnp // ns,), f32),
        ],
    )
    inv = hist(dst, et)

    bn = 400 if n % 400 == 0 else 100
    y1 = _transform(x, w1_ext, bn)

    common_scratch = [
        pltpu.VMEM((KCH, c), f32),
        pltpu.VMEM((KCH,), i32),
        pltpu.VMEM((KCH,), i32),
        pltpu.VMEM((KCH,), i32),
        pltpu.VMEM((KCH,), i32),
        pltpu.VMEM((KCH,), i32),
        pltpu.VMEM((KCH,), f32),
    ]

    value = pl.kernel(
        functools.partial(_sc_value_body, n=n, np_=np_, rnp=rnp,
                          nchunks=nchunks, c=c, nc=nc, ns=ns),
        out_type=jax.ShapeDtypeStruct((2, np_, c), f32),
        mesh=mesh,
        scratch_types=[
            pltpu.VMEM_SHARED((rnp,), f32),
            pltpu.VMEM_SHARED((np_, c), f32),
            *common_scratch,
            pltpu.VMEM((rnp // ns,), f32),
            pltpu.SemaphoreType.DMA,
        ],
    )

    acc1 = value(src, dst, et, y1, inv)
    y2 = _mid(acc1, y1, b1, gamma1, beta1, w2_ext, n, bn)
    acc2 = value(src, dst, et, y2, inv)

    return _final(acc2, y2, b2, n, bn)


# speedup vs baseline: 1.3647x; 1.0229x over previous
import functools

import jax
import jax.numpy as jnp
from jax import lax
from jax.experimental import pallas as pl
from jax.experimental.pallas import tpu as pltpu
from jax.experimental.pallas import tpu_sc as plsc

KCH = 128
LN_EPS = 1e-5


def _transform_body(x_ref, w_ref, y_ref):
    y_ref[...] = jnp.dot(x_ref[...], w_ref[0],
                         preferred_element_type=jnp.float32)


def _transform(x, w_ext, bn):
    n, c = x.shape
    k = w_ext.shape[0]
    nb = n // bn
    return pl.pallas_call(
        _transform_body,
        grid=(nb, k),
        in_specs=[
            pl.BlockSpec((bn, c), lambda i, r: (i, 0)),
            pl.BlockSpec((1, c, c), lambda i, r: (r, 0, 0)),
        ],
        out_specs=pl.BlockSpec((bn, c), lambda i, r, _nb=nb: (r * _nb + i, 0)),
        out_shape=jax.ShapeDtypeStruct((k * n, c), jnp.float32),
    )(x, w_ext)


def _mid_body(acc_ref, r1_ref, b_ref, g_ref, be_ref, w_ref, y_ref):
    s = acc_ref[0] + acc_ref[1] + r1_ref[...] + b_ref[...]
    mu = jnp.mean(s, axis=-1, keepdims=True)
    var = jnp.mean((s - mu) ** 2, axis=-1, keepdims=True)
    h = (s - mu) / jnp.sqrt(var + LN_EPS) * g_ref[...] + be_ref[...]
    h = jnp.maximum(h, 0.0)
    y_ref[...] = jnp.dot(h, w_ref[0], preferred_element_type=jnp.float32)


def _mid(acc, y1, b1, g1, be1, w2_ext, n, bn):
    c = y1.shape[1]
    k = w2_ext.shape[0]
    nb = n // bn
    return pl.pallas_call(
        _mid_body,
        grid=(nb, k),
        in_specs=[
            pl.BlockSpec((2, bn, c), lambda i, r: (0, i, 0)),
            pl.BlockSpec((bn, c), lambda i, r, _nb=nb, _k=k: ((_k - 1) * _nb + i, 0)),
            pl.BlockSpec((c,), lambda i, r: (0,)),
            pl.BlockSpec((c,), lambda i, r: (0,)),
            pl.BlockSpec((c,), lambda i, r: (0,)),
            pl.BlockSpec((1, c, c), lambda i, r: (r, 0, 0)),
        ],
        out_specs=pl.BlockSpec((bn, c), lambda i, r, _nb=nb: (r * _nb + i, 0)),
        out_shape=jax.ShapeDtypeStruct((k * n, c), jnp.float32),
    )(acc, y1, b1, g1, be1, w2_ext)


def _final_body(acc_ref, r2_ref, b_ref, out_ref):
    out_ref[...] = acc_ref[0] + acc_ref[1] + r2_ref[...] + b_ref[...]


def _final(acc, y2, b2, n, bn):
    c = y2.shape[1]
    k = y2.shape[0] // n
    nb = n // bn
    return pl.pallas_call(
        _final_body,
        grid=(nb,),
        in_specs=[
            pl.BlockSpec((2, bn, c), lambda i: (0, i, 0)),
            pl.BlockSpec((bn, c), lambda i, _nb=nb, _k=k: ((_k - 1) * _nb + i, 0)),
            pl.BlockSpec((c,), lambda i: (0,)),
        ],
        out_specs=pl.BlockSpec((bn, c), lambda i: (i, 0)),
        out_shape=jax.ShapeDtypeStruct((n, c), jnp.float32),
    )(acc, y2, b2)


def _zero_vec(ref, nwords):
    def st(j, _):
        ref[pl.ds(j * 16, 16)] = jnp.zeros((16,), jnp.float32)
        return 0
    lax.fori_loop(0, nwords // 16, st, 0)


def _fill_ones(ref, nwords):
    def st(j, _):
        ref[pl.ds(j * 16, 16)] = jnp.ones((16,), jnp.float32)
        return 0
    lax.fori_loop(0, nwords // 16, st, 0)


def _agg_value_pass(src_hbm, dst_hbm, et_hbm, y_hbm, acc_sh, inv_sh,
                    rows_v, sbuf, dbuf, tbuf, ibuf, widx, wbuf, sem,
                    wid, nw, n, nchunks, c):
    nk = nchunks // nw
    cl = c // 16

    def chunk(k_, _):
        base = (wid + k_ * nw) * KCH
        pltpu.sync_copy(src_hbm.at[pl.ds(base, KCH)], sbuf)
        pltpu.sync_copy(dst_hbm.at[pl.ds(base, KCH)], dbuf)
        pltpu.sync_copy(et_hbm.at[pl.ds(base, KCH)], tbuf)

        def gw(j, _):
            sl = pl.ds(j * 16, 16)
            t = tbuf[sl]
            ibuf[sl] = t * n + sbuf[sl]
            widx[sl] = t * n + dbuf[sl]
            return 0
        lax.fori_loop(0, KCH // 16, gw, 0)

        pltpu.async_copy(y_hbm.at[ibuf], rows_v, sem).wait()
        pltpu.async_copy(inv_sh.at[widx], wbuf, sem).wait()

        def scale(g, _):
            w16 = wbuf[pl.ds(g * 16, 16)]
            for e_ in range(16):
                i = g * 16 + e_
                w = w16[e_]
                for j in range(cl):
                    sl = pl.ds(j * 16, 16)
                    rows_v[i, sl] = rows_v[i, sl] * w
            return 0
        lax.fori_loop(0, KCH // 16, scale, 0)

        pltpu.sync_copy(rows_v, acc_sh.at[dbuf], add=True)
        return 0

    lax.fori_loop(0, nk, chunk, 0)


def _acc_writeout(acc_sh, rows_v, acc_hbm, core, sid, asl, c):
    for z in range(asl // KCH):
        row = sid * asl + z * KCH
        pltpu.sync_copy(acc_sh.at[pl.ds(row, KCH)], rows_v)
        pltpu.sync_copy(rows_v, acc_hbm.at[core, pl.ds(row, KCH)])


def _sc_hist_body(dst_hbm, et_hbm, cnt_hbm, cnt_sh, dbuf, tbuf, ibuf,
                  ones_v, zbuf, *, n, rnp, nchunks, nc, ns):
    core = lax.axis_index("c")
    sid = lax.axis_index("s")
    csl = rnp // ns

    _zero_vec(zbuf, csl)
    _fill_ones(ones_v, KCH)
    tsl = pl.ds(sid * csl, csl)
    pltpu.sync_copy(zbuf, cnt_sh.at[tsl])
    plsc.subcore_barrier()

    nk_c = nchunks // (nc * ns)
    half = nchunks // nc

    def count_chunk(k_, _):
        base = (core * half + sid + k_ * ns) * KCH
        pltpu.sync_copy(dst_hbm.at[pl.ds(base, KCH)], dbuf)
        pltpu.sync_copy(et_hbm.at[pl.ds(base, KCH)], tbuf)

        def cidx(j, _):
            sl = pl.ds(j * 16, 16)
            ibuf[sl] = tbuf[sl] * n + dbuf[sl]
            return 0
        lax.fori_loop(0, KCH // 16, cidx, 0)
        pltpu.sync_copy(ones_v, cnt_sh.at[ibuf], add=True)
        return 0

    lax.fori_loop(0, nk_c, count_chunk, 0)
    plsc.subcore_barrier()

    pltpu.sync_copy(cnt_sh.at[tsl], zbuf)
    pltpu.sync_copy(zbuf, cnt_hbm.at[pl.ds(core * rnp + sid * csl, csl)])


def _sc_value_body(src_hbm, dst_hbm, et_hbm, y_hbm, cnt_hbm, acc_hbm,
                   inv_sh, acc_sh, rows_v, sbuf, dbuf, tbuf, ibuf, widx,
                   wbuf, cbuf, cb2, sem, *, n, np_, rnp, nchunks, c, nc, ns):
    core = lax.axis_index("c")
    sid = lax.axis_index("s")
    wid = core * ns + sid
    nw = nc * ns
    asl = np_ // ns
    csl = rnp // ns

    def zr(i, _):
        for j in range(c // 16):
            rows_v[i, pl.ds(j * 16, 16)] = jnp.zeros((16,), jnp.float32)
        return 0
    lax.fori_loop(0, KCH, zr, 0)
    for z in range(asl // KCH):
        pltpu.sync_copy(rows_v, acc_sh.at[pl.ds(sid * asl + z * KCH, KCH)])
    tsl = pl.ds(sid * csl, csl)
    pltpu.sync_copy(cnt_hbm.at[pl.ds(sid * csl, csl)], cbuf)
    pltpu.sync_copy(cnt_hbm.at[pl.ds(rnp + sid * csl, csl)], cb2)

    def invb(j, _):
        sl = pl.ds(j * 16, 16)
        cbuf[sl] = 1.0 / jnp.maximum(cbuf[sl] + cb2[sl], 1.0)
        return 0
    lax.fori_loop(0, csl // 16, invb, 0)
    pltpu.sync_copy(cbuf, inv_sh.at[tsl])
    plsc.subcore_barrier()

    _agg_value_pass(src_hbm, dst_hbm, et_hbm, y_hbm, acc_sh, inv_sh,
                    rows_v, sbuf, dbuf, tbuf, ibuf, widx, wbuf, sem,
                    wid, nw, n, nchunks, c)
    plsc.subcore_barrier()

    _acc_writeout(acc_sh, rows_v, acc_hbm, core, sid, asl, c)


def kernel(x, edge_index, edge_type, W1, root1, b1, gamma1, beta1,
           W2, root2, b2):
    n, c = x.shape
    e = edge_type.shape[0]
    r = W1.shape[0]

    info = plsc.get_sparse_core_info()
    nc, ns = info.num_cores, info.num_subcores
    nw = nc * ns

    asl = ((-(-n // ns)) + KCH - 1) // KCH * KCH
    np_ = asl * ns
    csl = (-(-(r * n) // ns) + 15) // 16 * 16
    rnp = csl * ns
    nchunks = -(-e // (nw * KCH)) * nw
    e_pad = nchunks * KCH

    src = edge_index[0].astype(jnp.int32)
    dst = edge_index[1].astype(jnp.int32)
    et = edge_type.astype(jnp.int32)
    if e_pad > e:
        pad = e_pad - e
        src = jnp.concatenate([src, jnp.zeros((pad,), jnp.int32)])
        dst = jnp.concatenate([dst, jnp.full((pad,), n, jnp.int32)])
        et = jnp.concatenate([et, jnp.full((pad,), r - 1, jnp.int32)])

    w1_ext = jnp.concatenate([W1, root1[None]], axis=0)
    w2_ext = jnp.concatenate([W2, root2[None]], axis=0)

    mesh = plsc.VectorSubcoreMesh(core_axis_name="c", subcore_axis_name="s")
    f32 = jnp.float32
    i32 = jnp.int32

    hist = pl.kernel(
        functools.partial(_sc_hist_body, n=n, rnp=rnp, nchunks=nchunks,
                          nc=nc, ns=ns),
        out_type=jax.ShapeDtypeStruct((2 * rnp,), f32),
        mesh=mesh,
        scratch_types=[
            pltpu.VMEM_SHARED((rnp,), f32),
            pltpu.VMEM((KCH,), i32),
            pltpu.VMEM((KCH,), i32),
            pltpu.VMEM((KCH,), i32),
            pltpu.VMEM((KCH,), f32),
            pltpu.VMEM((rnp // ns,), f32),
        ],
    )
    cnts = hist(dst, et)

    bn = 400 if n % 400 == 0 else 100
    y1 = _transform(x, w1_ext, bn)

    common_scratch = [
        pltpu.VMEM((KCH, c), f32),
        pltpu.VMEM((KCH,), i32),
        pltpu.VMEM((KCH,), i32),
        pltpu.VMEM((KCH,), i32),
        pltpu.VMEM((KCH,), i32),
        pltpu.VMEM((KCH,), i32),
        pltpu.VMEM((KCH,), f32),
    ]

    value = pl.kernel(
        functools.partial(_sc_value_body, n=n, np_=np_, rnp=rnp,
                          nchunks=nchunks, c=c, nc=nc, ns=ns),
        out_type=jax.ShapeDtypeStruct((2, np_, c), f32),
        mesh=mesh,
        scratch_types=[
            pltpu.VMEM_SHARED((rnp,), f32),
            pltpu.VMEM_SHARED((np_, c), f32),
            *common_scratch,
            pltpu.VMEM((rnp // ns,), f32),
            pltpu.VMEM((rnp // ns,), f32),
            pltpu.SemaphoreType.DMA,
        ],
    )

    acc1 = value(src, dst, et, y1, cnts)
    y2 = _mid(acc1, y1, b1, gamma1, beta1, w2_ext, n, bn)
    acc2 = value(src, dst, et, y2, cnts)

    return _final(acc2, y2, b2, n, bn)
